# baseline (speedup 1.0000x reference)
import functools

import numpy as np
import jax
from jax import lax
import jax.numpy as jnp
from jax.experimental import pallas as pl
from jax.experimental.pallas import tpu as pltpu
from jax.experimental.pallas import tpu_sc as plsc

NUM_C_TYPES = 6
TTYPE_UNKNOWN = 5
E = 8
D = 768
H = D // 4
TEMP = 0.1
FLOOR = min(0.05, 0.15 / 4)

TBLK = 2048
TPW = 1024
NW = 32
L = 16
OBLK = 8192

BIG = np.float32(1e20)


def _mlp_body(x_ref, W1_ref, b1_ref, W2_ref, b2_ref, out_ref):
    x = x_ref[0]
    u = jax.lax.dot_general(x, W1_ref[...], (((1,), (0,)), ((), ())),
                            preferred_element_type=jnp.float32) + b1_ref[...]
    h = 0.5 * u * (1.0 + jax.lax.erf(u * np.float32(1.0 / np.sqrt(2.0))))
    logitsT = (jax.lax.dot_general(W2_ref[...], h, (((0,), (1,)), ((), ())),
                                   preferred_element_type=jnp.float32)
               + b2_ref[...]) * (1.0 / TEMP)
    z = logitsT - jnp.max(logitsT, axis=0, keepdims=True)
    ez = jnp.exp(z)
    p2 = ez / jnp.sum(ez, axis=0, keepdims=True)
    for s in range(TBLK // TPW):
        out_ref[s] = p2[:, s * TPW:(s + 1) * TPW]


def _mlp_p2(tokens, W1, b1, W2, b2):
    B, N, _ = tokens.shape
    nblk = N // TBLK
    slabs = TBLK // TPW
    return pl.pallas_call(
        _mlp_body,
        grid=(B, nblk),
        in_specs=[
            pl.BlockSpec((1, TBLK, D), lambda b, j: (b, j, 0)),
            pl.BlockSpec((D, H), lambda b, j: (0, 0)),
            pl.BlockSpec((1, H), lambda b, j: (0, 0)),
            pl.BlockSpec((H, E), lambda b, j: (0, 0)),
            pl.BlockSpec((E, 1), lambda b, j: (0, 0)),
        ],
        out_specs=pl.BlockSpec((slabs, E, TPW),
                               lambda b, j, _n=nblk: (b * _n + j, 0, 0)),
        out_shape=jax.ShapeDtypeStruct((NW, E, TPW), jnp.float32),
        compiler_params=pltpu.CompilerParams(
            dimension_semantics=("parallel", "parallel")),
    )(tokens, W1, b1.reshape(1, H), W2, b2.reshape(E, 1))


def _make_sc_route(BN):
    mesh = plsc.VectorSubcoreMesh(core_axis_name="c", subcore_axis_name="s")

    @functools.partial(
        pl.kernel, mesh=mesh,
        out_type=jax.ShapeDtypeStruct((BN,), jnp.float32),
        scratch_types=[
            pltpu.VMEM((E, TPW), jnp.float32),
            pltpu.VMEM((TPW,), jnp.float32),
            pltpu.VMEM((TPW,), jnp.float32),
        ],
    )
    def sc_route(p2_hbm, ttf_hbm, am_hbm, p2_v, ttf_v, am_v):
        wid = lax.axis_index("s") * 2 + lax.axis_index("c")
        base = wid * TPW
        pltpu.sync_copy(ttf_hbm.at[pl.ds(base, TPW)], ttf_v)
        pltpu.sync_copy(p2_hbm.at[wid], p2_v)

        alpha = min(FLOOR * E, 1.0)
        c1 = np.float32(1.0 - alpha)
        c2 = np.float32(alpha / E)

        for c in range(TPW // L):
            off = c * L
            ttf = ttf_v[pl.ds(off, L)]
            rows = [c1 * p2_v[e, pl.ds(off, L)] + c2 for e in range(E)]
            m = rows[0]
            for e in range(1, E):
                m = jnp.maximum(m, rows[e])
            am = jnp.minimum((m - rows[0]) * BIG, np.float32(1.0)) * 8.0
            for e in range(1, E):
                ke = (jnp.minimum((m - rows[e]) * BIG, np.float32(1.0))
                      * 8.0 + np.float32(e))
                am = jnp.minimum(am, ke)
            u = jnp.minimum((np.float32(5.0) - ttf) * BIG, np.float32(1.0))
            am = u * ttf + (np.float32(1.0) - u) * am
            am_v[pl.ds(off, L)] = am

        pltpu.sync_copy(am_v, am_hbm.at[pl.ds(base, TPW)])

    return sc_route


def _onehot_body(am_ref, out_ref):
    am = am_ref[...]
    ii = lax.broadcasted_iota(jnp.int32, (OBLK, E), 1).astype(jnp.float32)
    out_ref[...] = (ii == am).astype(jnp.float32)


def _expand_onehot(amF, B, N):
    BN = B * N
    return pl.pallas_call(
        _onehot_body,
        grid=(BN // OBLK,),
        in_specs=[pl.BlockSpec((OBLK, 1), lambda i: (i, 0))],
        out_specs=pl.BlockSpec((OBLK, E), lambda i: (i, 0)),
        out_shape=jax.ShapeDtypeStruct((BN, E), jnp.float32),
        compiler_params=pltpu.CompilerParams(
            dimension_semantics=("parallel",)),
    )(amF.reshape(BN, 1))


@jax.jit
def _route_all(tokens, token_types, W1, b1, W2, b2):
    B, N, _ = tokens.shape
    BN = B * N
    p2 = _mlp_p2(tokens, W1, b1, W2, b2)
    ttf = token_types.reshape(BN).astype(jnp.float32)
    amF = _make_sc_route(BN)(p2, ttf)
    onehot = _expand_onehot(amF, B, N)
    return onehot.reshape(B, N, E)


def kernel(tokens, token_types, t, W1, b1, W2, b2):
    onehot = _route_all(tokens, token_types, W1, b1, W2, b2)
    return onehot, onehot


# speedup vs baseline: 1.6529x; 1.6529x over previous
import functools

import numpy as np
import jax
import jax.numpy as jnp
from jax.experimental import pallas as pl
from jax.experimental.pallas import tpu as pltpu

NUM_C_TYPES = 6
TTYPE_UNKNOWN = 5
E = 8
D = 768
H = D // 4
TEMP = 0.1
SOFT_RES = 0.05
FLOOR = min(0.05, 0.15 / 4)
CAP_LOW = 0.5
CAP_HIGH = 0.6
T_MAX = 1000

TBLK = 4096


def _base_table():
    base = np.zeros((NUM_C_TYPES, E), dtype=np.float32)
    for t_type in range(NUM_C_TYPES - 1):
        base[t_type, t_type % E] = 1.0
    base[TTYPE_UNKNOWN] = 1.0 / E
    num_known = NUM_C_TYPES - 1
    if E > num_known:
        for extra_idx in range(num_known, E):
            paired = extra_idx % num_known
            base[paired, extra_idx] = 0.3
        for t_type in range(NUM_C_TYPES - 1):
            s = base[t_type].sum()
            if s > 0:
                base[t_type] = base[t_type] / s
    return base


def _router_body(t_ref, x_ref, tt_ref, W1_ref, b1_ref, W2_ref, b2_ref,
                 baseT_ref, eye_ref, out_ref):
    x = x_ref[0]
    tt = tt_ref[0, 0]

    u = jax.lax.dot_general(x, W1_ref[...], (((1,), (0,)), ((), ())),
                            preferred_element_type=jnp.float32) + b1_ref[...]
    h = 0.5 * u * (1.0 + jax.lax.erf(u * np.float32(1.0 / np.sqrt(2.0))))

    logitsT = (jax.lax.dot_general(W2_ref[...], h, (((0,), (1,)), ((), ())),
                                   preferred_element_type=jnp.float32)
               + b2_ref[...]) * (1.0 / TEMP)

    z = logitsT - jnp.max(logitsT, axis=0, keepdims=True)
    ez = jnp.exp(z)
    p2 = ez / jnp.sum(ez, axis=0, keepdims=True)

    baseT = baseT_ref[...]
    bp = jnp.zeros_like(p2)
    for k in range(NUM_C_TYPES):
        col = jax.lax.slice(baseT, (0, k), (E, k + 1))
        bp = bp + jnp.where(tt == k, col, 0.0)

    w = jnp.where(tt == TTYPE_UNKNOWN, 0.0, 1.0 - SOFT_RES)
    blended = w * bp + (1.0 - w) * p2

    alpha = min(FLOOR * E, 1.0)
    probs = (1.0 - alpha) * blended + alpha / E

    b = pl.program_id(0)
    t_norm = t_ref[b].astype(jnp.float32) / T_MAX
    cap = CAP_LOW + (CAP_HIGH + CAP_LOW) * t_norm

    excess = jnp.maximum(probs - cap, 0.0)
    capped = probs - excess
    headroom = jnp.maximum(cap - capped, 0.0)
    hs = jnp.maximum(jnp.sum(headroom, axis=0, keepdims=True), 1e-8)
    final = capped + jnp.sum(excess, axis=0, keepdims=True) * (headroom / hs)

    m = jnp.max(final, axis=0, keepdims=True)
    idx = jax.lax.broadcasted_iota(jnp.int32, final.shape, 0)
    cand = jnp.where(final >= m, idx, E)
    amin = jnp.min(cand, axis=0, keepdims=True)
    yT = (idx == amin).astype(jnp.float32)

    out_ref[0] = jax.lax.dot_general(yT, eye_ref[...], (((0,), (0,)), ((), ())),
                                     preferred_element_type=jnp.float32)


@jax.jit
def _router(tokens, token_types, t, W1, b1, W2, b2, baseT, eye):
    B, N, _ = tokens.shape
    nblk = N // TBLK
    tt4 = token_types.reshape(B, nblk, 1, TBLK)
    grid = (B, nblk)
    onehot = pl.pallas_call(
        _router_body,
        grid=grid,
        in_specs=[
            pl.BlockSpec(memory_space=pltpu.SMEM),
            pl.BlockSpec((1, TBLK, D), lambda b, j: (b, j, 0)),
            pl.BlockSpec((1, 1, 1, TBLK), lambda b, j: (b, j, 0, 0)),
            pl.BlockSpec((D, H), lambda b, j: (0, 0)),
            pl.BlockSpec((1, H), lambda b, j: (0, 0)),
            pl.BlockSpec((H, E), lambda b, j: (0, 0)),
            pl.BlockSpec((E, 1), lambda b, j: (0, 0)),
            pl.BlockSpec((E, NUM_C_TYPES), lambda b, j: (0, 0)),
            pl.BlockSpec((E, E), lambda b, j: (0, 0)),
        ],
        out_specs=pl.BlockSpec((1, TBLK, E), lambda b, j: (b, j, 0)),
        out_shape=jax.ShapeDtypeStruct((B, N, E), jnp.float32),
        compiler_params=pltpu.CompilerParams(
            dimension_semantics=("parallel", "parallel")),
    )(t, tokens, tt4, W1, b1.reshape(1, H), W2, b2.reshape(E, 1),
      baseT, eye)
    return onehot


def kernel(tokens, token_types, t, W1, b1, W2, b2):
    baseT = jnp.asarray(_base_table().T.copy())
    eye = jnp.eye(E, dtype=jnp.float32)
    onehot = _router(tokens, token_types, t, W1, b1, W2, b2, baseT, eye)
    return onehot, onehot
